# baseline (speedup 1.0000x reference)
import functools

import jax
import jax.numpy as jnp
from jax import lax
from jax.experimental import pallas as pl
from jax.experimental.pallas import tpu as pltpu
from jax.experimental.pallas import tpu_sc as plsc

NUM_EMB = 1000000
FEAT = 32
N_B = 16384
N_J = 26
B_TOTAL = N_B * N_J
IDX_W = 128
N_ROWS = B_TOTAL // IDX_W
NC, NS = 2, 16
NW = NC * NS
B_BLK = 512
GPT = B_BLK // IDX_W
N_TASKS = N_J * (N_B // B_BLK)
TPW = N_TASKS // NW
K = TPW * GPT
N_BLK = N_B // B_BLK


@functools.partial(
    pl.kernel,
    out_type=jax.ShapeDtypeStruct((N_J, FEAT, N_B), jnp.float32),
    mesh=plsc.VectorSubcoreMesh(core_axis_name="c", subcore_axis_name="s"),
    scratch_types=[
        pltpu.VMEM((K, IDX_W), jnp.int32),
        pltpu.VMEM((2, B_BLK, FEAT), jnp.float32),
        pltpu.VMEM((2, FEAT, B_BLK), jnp.float32),
        pltpu.SemaphoreType.DMA,
        pltpu.SemaphoreType.DMA,
        pltpu.SemaphoreType.DMA,
    ],
    compiler_params=pltpu.CompilerParams(
        use_tc_tiling_on_sc=False, needs_layout_passes=False
    ),
)
def _embed_sc(
    idx_hbm, table_hbm, out_hbm, idx_v, rows_v, tr_v, sem_g, sem_o0, sem_o1
):
    wid = lax.axis_index("s") * NC + lax.axis_index("c")
    pltpu.sync_copy(idx_hbm.at[pl.ds(wid * K, K)], idx_v)
    t0 = wid * TPW
    sems = (sem_o0, sem_o1)

    def fire(i, buf):
        for r in range(GPT):
            pltpu.async_copy(
                table_hbm.at[idx_v.at[i * GPT + r]],
                rows_v.at[buf, pl.ds(r * IDX_W, IDX_W)],
                sem_g,
            )

    def drain_gathers():
        pltpu.make_async_copy(
            table_hbm.at[pl.ds(0, B_BLK)], rows_v.at[0], sem_g
        ).wait()

    f_lo = lax.iota(jnp.int32, 16)
    f_hi = f_lo + 16

    def transpose(buf):
        def body(b, carry):
            b_sp = jnp.full((16,), b, jnp.int32)
            x0 = rows_v[buf, b, pl.ds(0, 16)]
            x1 = rows_v[buf, b, pl.ds(16, 16)]
            plsc.store_scatter(tr_v.at[buf], [f_lo, b_sp], x0)
            plsc.store_scatter(tr_v.at[buf], [f_hi, b_sp], x1)
            return carry

        lax.fori_loop(0, B_BLK, body, 0)

    def start_out(i, buf):
        t = t0 + i
        j = t // N_BLK
        blk = t % N_BLK
        pltpu.async_copy(
            tr_v.at[buf],
            out_hbm.at[j, :, pl.ds(blk * B_BLK, B_BLK)],
            sems[buf],
        )

    def drain_out(buf):
        pltpu.make_async_copy(
            tr_v.at[0], out_hbm.at[0, :, pl.ds(0, B_BLK)], sems[buf]
        ).wait()

    fire(0, 0)
    drain_gathers()
    fire(1, 1)
    transpose(0)
    start_out(0, 0)

    drain_gathers()
    fire(2, 0)
    transpose(1)
    start_out(1, 1)

    def pair(p, carry):
        i = 2 * p + 2
        drain_gathers()
        fire(i + 1, 1)
        drain_out(0)
        transpose(0)
        start_out(i, 0)

        drain_gathers()
        fire(i + 2, 0)
        drain_out(1)
        transpose(1)
        start_out(i + 1, 1)
        return carry

    lax.fori_loop(0, (TPW - 4) // 2, pair, 0)

    drain_gathers()
    fire(TPW - 1, 1)
    drain_out(0)
    transpose(0)
    start_out(TPW - 2, 0)

    drain_gathers()
    drain_out(1)
    transpose(1)
    start_out(TPW - 1, 1)

    drain_out(0)
    drain_out(1)


def kernel(inputs, embedding):
    idx2d = inputs.T.reshape(N_ROWS, IDX_W)
    out = _embed_sc(idx2d, embedding)
    return jnp.transpose(out, (2, 0, 1))
